# baseline (speedup 1.0000x reference)
import functools

import jax
import jax.numpy as jnp
from jax import lax
from jax.experimental import pallas as pl
from jax.experimental.pallas import tpu as pltpu
from jax.experimental.pallas import tpu_sc as plsc

_BUCKETS = 32
_DIM = 128
_BATCH = 4096
_SEQ = 50
_NTILES = 32
_BPT = _BATCH // _NTILES
_NGRP = _BPT // 16


def _fuse_body(q_ref, r_ref, qr_ref):
    qr_ref[pl.ds(0, _BUCKETS), :] = q_ref[...]
    qr_ref[pl.ds(_BUCKETS, _BUCKETS), :] = r_ref[...]


def _fuse_tables(q, r):
    return pl.pallas_call(
        _fuse_body,
        out_shape=jax.ShapeDtypeStruct((2 * _BUCKETS, _DIM), jnp.float32),
    )(q, r)


def _lookup_body(qr_hbm, idxt_hbm, out_hbm, qr_t, idx_v, bufs, sem_w):
    wid = lax.axis_index("s") * 2 + lax.axis_index("c")
    b0 = wid * _BPT
    pltpu.sync_copy(qr_hbm, qr_t)
    pltpu.sync_copy(idxt_hbm.at[:, pl.ds(b0, _BPT)], idx_v)

    iota = lax.iota(jnp.int32, 16)

    def fill(s, p):
        bufp = bufs.at[p]

        @pl.loop(0, _NGRP)
        def _grp(g):
            v = idx_v[s, pl.ds(g * 16, 16)]
            vq = v >> 5
            vr = (v & (_BUCKETS - 1)) + _BUCKETS
            rows = g * 16 + iota
            for d in range(_DIM):
                cold = jnp.full((16,), d, jnp.int32)
                xq = plsc.load_gather(qr_t, [vq, cold])
                xr = plsc.load_gather(qr_t, [vr, cold])
                plsc.store_scatter(bufp, [rows, cold], xq * xr)

    def write(s, p):
        pltpu.async_copy(
            bufs.at[p],
            out_hbm.at[s, pl.ds(b0, _BPT)],
            sem_w.at[p],
        )

    def wait_w(p):
        pltpu.make_async_copy(
            bufs.at[p],
            out_hbm.at[0, pl.ds(b0, _BPT)],
            sem_w.at[p],
        ).wait()

    fill(0, 0)
    write(0, 0)
    fill(1, 1)
    write(1, 1)

    @pl.loop(1, _SEQ // 2)
    def _pair(t):
        for p in range(2):
            s = t * 2 + p
            wait_w(p)
            fill(s, p)
            write(s, p)

    wait_w(0)
    wait_w(1)


def _sc_lookup(qr, idx_t):
    mesh = plsc.VectorSubcoreMesh(core_axis_name="c", subcore_axis_name="s")
    return pl.kernel(
        _lookup_body,
        out_type=jax.ShapeDtypeStruct((_SEQ, _BATCH, _DIM), jnp.float32),
        mesh=mesh,
        compiler_params=pltpu.CompilerParams(needs_layout_passes=False),
        scratch_types=[
            pltpu.VMEM((2 * _BUCKETS, _DIM), jnp.float32),
            pltpu.VMEM((_SEQ, _BPT), jnp.int32),
            pltpu.VMEM((2, _BPT, _DIM), jnp.float32),
            pltpu.SemaphoreType.DMA((2,)),
        ],
    )(qr, idx_t)


@jax.jit
def kernel(inputs, q_embeddings, r_embeddings):
    qr = _fuse_tables(q_embeddings, r_embeddings)
    out = _sc_lookup(qr, inputs.T)
    return out.transpose(1, 0, 2)


# speedup vs baseline: 1.5196x; 1.5196x over previous
import functools

import jax
import jax.numpy as jnp
from jax import lax
from jax.experimental import pallas as pl
from jax.experimental.pallas import tpu as pltpu
from jax.experimental.pallas import tpu_sc as plsc

_BUCKETS = 32
_DIM = 128
_BATCH = 4096
_SEQ = 50
_NTILES = 32
_BPT = _BATCH // _NTILES
_NGRP = _BPT // 16


def _fuse_body(q_ref, r_ref, qr_ref):
    qr_ref[pl.ds(0, _BUCKETS), :] = q_ref[...]
    qr_ref[pl.ds(_BUCKETS, _BUCKETS), :] = r_ref[...]


def _fuse_tables(q, r):
    return pl.pallas_call(
        _fuse_body,
        out_shape=jax.ShapeDtypeStruct((2 * _BUCKETS, _DIM), jnp.float32),
    )(q, r)


def _lookup_body(qr_hbm, idxt_hbm, out_hbm, qr_t, idx_v, bufs, sem_w):
    wid = lax.axis_index("s") * 2 + lax.axis_index("c")
    b0 = wid * _BPT
    pltpu.sync_copy(qr_hbm, qr_t)
    pltpu.sync_copy(idxt_hbm.at[:, pl.ds(b0, _BPT)], idx_v)

    iota = lax.iota(jnp.int32, 16)

    def fill(s, p):
        bufp = bufs.at[p]

        @pl.loop(0, _NGRP)
        def _grp(g):
            v = idx_v[s, pl.ds(g * 16, 16)]
            vq = v >> 5
            vr = (v & (_BUCKETS - 1)) + _BUCKETS
            rows = g * 16 + iota
            for db in range(0, _DIM, 8):
                colds = [jnp.full((16,), db + k, jnp.int32) for k in range(8)]
                xqs = [plsc.load_gather(qr_t, [vq, colds[k]]) for k in range(8)]
                xrs = [plsc.load_gather(qr_t, [vr, colds[k]]) for k in range(8)]
                for k in range(8):
                    plsc.store_scatter(bufp, [rows, colds[k]], xqs[k] * xrs[k])

    def write(s, p):
        pltpu.async_copy(
            bufs.at[p],
            out_hbm.at[s, pl.ds(b0, _BPT)],
            sem_w.at[p],
        )

    def wait_w(p):
        pltpu.make_async_copy(
            bufs.at[p],
            out_hbm.at[0, pl.ds(b0, _BPT)],
            sem_w.at[p],
        ).wait()

    fill(0, 0)
    write(0, 0)
    fill(1, 1)
    write(1, 1)

    @pl.loop(1, _SEQ // 2)
    def _pair(t):
        for p in range(2):
            s = t * 2 + p
            wait_w(p)
            fill(s, p)
            write(s, p)

    wait_w(0)
    wait_w(1)


def _sc_lookup(qr, idx_t):
    mesh = plsc.VectorSubcoreMesh(core_axis_name="c", subcore_axis_name="s")
    return pl.kernel(
        _lookup_body,
        out_type=jax.ShapeDtypeStruct((_SEQ, _BATCH, _DIM), jnp.float32),
        mesh=mesh,
        compiler_params=pltpu.CompilerParams(needs_layout_passes=False),
        scratch_types=[
            pltpu.VMEM((2 * _BUCKETS, _DIM), jnp.float32),
            pltpu.VMEM((_SEQ, _BPT), jnp.int32),
            pltpu.VMEM((2, _BPT, _DIM), jnp.float32),
            pltpu.SemaphoreType.DMA((2,)),
        ],
    )(qr, idx_t)


@jax.jit
def kernel(inputs, q_embeddings, r_embeddings):
    qr = _fuse_tables(q_embeddings, r_embeddings)
    out = _sc_lookup(qr, inputs.T)
    return out.transpose(1, 0, 2)


# speedup vs baseline: 10.5589x; 6.9486x over previous
import functools

import jax
import jax.numpy as jnp
from jax import lax
from jax.experimental import pallas as pl
from jax.experimental.pallas import tpu as pltpu
from jax.experimental.pallas import tpu_sc as plsc

_BUCKETS = 32
_DIM = 128
_CROWS = _BUCKETS * _BUCKETS
_BATCH = 4096
_SEQ = 50
_NTILES = 32
_BPT = _BATCH // _NTILES
_CB = 4
_NCHUNK = _BPT // _CB


def _build_c_body(q_ref, r_ref, c_ref):
    r_all = r_ref[...]

    @pl.loop(0, _BUCKETS)
    def _row(i):
        c_ref[pl.ds(i * _BUCKETS, _BUCKETS), :] = q_ref[pl.ds(i, 1), :] * r_all


def _combined_table(q, r):
    return pl.pallas_call(
        _build_c_body,
        out_shape=jax.ShapeDtypeStruct((_CROWS, _DIM), jnp.float32),
    )(q, r)


_NPOOL = 6
_DEPTH = 4


def _gather_body(c_hbm, idxt_hbm, out_hbm, idx_v, bufs, sem_g, sem_w):
    wid = lax.axis_index("s") * 2 + lax.axis_index("c")
    b0 = wid * _BPT
    pltpu.sync_copy(idxt_hbm.at[:, pl.ds(b0, _BPT)], idx_v)

    def gather(s):
        return pltpu.async_copy(
            c_hbm.at[idx_v.at[s]], bufs.at[s % _NPOOL], sem_g.at[s % _NPOOL]
        )

    def write(s):
        return pltpu.async_copy(
            bufs.at[s % _NPOOL],
            out_hbm.at[s, pl.ds(b0, _BPT)],
            sem_w.at[s % _NPOOL],
        )

    gd, wd = {}, {}
    for s in range(_DEPTH):
        gd[s] = gather(s)
    for s in range(_SEQ):
        nxt = s + _DEPTH
        if nxt < _SEQ:
            if nxt - _NPOOL >= 0:
                wd[nxt - _NPOOL].wait()
            gd[nxt] = gather(nxt)
        gd[s].wait()
        wd[s] = write(s)
    for s in range(_SEQ - _NPOOL + _DEPTH, _SEQ):
        wd[s].wait()


def _sc_lookup(c, idx_t):
    mesh = plsc.VectorSubcoreMesh(core_axis_name="c", subcore_axis_name="s")
    return pl.kernel(
        _gather_body,
        out_type=jax.ShapeDtypeStruct((_SEQ, _BATCH, _DIM), jnp.float32),
        mesh=mesh,
        compiler_params=pltpu.CompilerParams(use_tc_tiling_on_sc=True),
        scratch_types=[
            pltpu.VMEM((_SEQ, _BPT), jnp.int32),
            pltpu.VMEM((_NPOOL, _BPT, _DIM), jnp.float32),
            pltpu.SemaphoreType.DMA((_NPOOL,)),
            pltpu.SemaphoreType.DMA((_NPOOL,)),
        ],
    )(c, idx_t)


@jax.jit
def kernel(inputs, q_embeddings, r_embeddings):
    c = _combined_table(q_embeddings, r_embeddings)
    out = _sc_lookup(c, inputs.T)
    return out.transpose(1, 0, 2)
